# baseline (speedup 1.0000x reference)
import dataclasses

import jax
import jax.numpy as jnp
from jax import lax
from jax.experimental import pallas as pl
from jax.experimental.pallas import tpu as pltpu
from jax.experimental.pallas import tpu_sc as plsc

N = 1048576
CH = 8
DIMS = (64, 64, 64, 32)
STRIDES = (DIMS[1] * DIMS[2] * DIMS[3], DIMS[2] * DIMS[3], DIMS[3], 1)
NW = 32
PPW = N // NW
B = 128
NBLK = PPW // B
NSG = B // 16
NROW = 8 * B
NSTREAM = NROW // 128


def _body(xs_hbm, data_hbm, out_hbm, xsb, idxb, wbuf, gbuf, outc, sem):
    wid = lax.axis_index("s") * 2 + lax.axis_index("c")
    base_pt = wid * PPW

    iota16 = lax.iota(jnp.int32, 16)
    cconst = [jnp.full((16,), c, jnp.int32) for c in range(2 * CH)]

    @pl.loop(0, NBLK)
    def _(blk):
        start = base_pt + blk * B
        for d in range(4):
            pltpu.sync_copy(xs_hbm.at[d, pl.ds(start, B)], xsb.at[d])

        @pl.loop(0, NSG)
        def _(sg):
            lo = []
            hi = []
            f = []
            for d in range(4):
                cd = xsb[d, pl.ds(sg * 16, 16)]
                half = 0.5 * DIMS[d]
                sd = cd * half + half
                idx_i = sd.astype(jnp.int32)
                f.append(sd - idx_i.astype(jnp.float32))
                lod = jnp.minimum(jnp.maximum(idx_i, 0), DIMS[d] - 1)
                lo.append(lod)
                hi.append(jnp.minimum(lod + 1, DIMS[d] - 1))
            px = [lo[0] * STRIDES[0], hi[0] * STRIDES[0]]
            py = [lo[1] * STRIDES[1], hi[1] * STRIDES[1]]
            pz = [
                ((lo[2] >> 3) << 8) + (lo[2] & 7),
                ((hi[2] >> 3) << 8) + (hi[2] & 7),
            ]
            t0 = lo[3] * 8
            wx = [1.0 - f[0], f[0]]
            wy = [1.0 - f[1], f[1]]
            wz = [1.0 - f[2], f[2]]
            wt = [1.0 - f[3], f[3]]
            pxy = {}
            wxy = {}
            for bx in range(2):
                for by in range(2):
                    pxy[(bx, by)] = px[bx] + py[by]
                    wxy[(bx, by)] = wx[bx] * wy[by]
            for b in range(8):
                bx, by, bz = b & 1, (b >> 1) & 1, (b >> 2) & 1
                wxyz = wxy[(bx, by)] * wz[bz]
                ofs = sg * 128 + b * 16
                idxb[pl.ds(ofs, 16)] = pxy[(bx, by)] + pz[bz] + t0
                wbuf[pl.ds(2 * ofs, 16)] = wxyz * wt[0]
                wbuf[pl.ds(2 * ofs + 16, 16)] = wxyz * wt[1]

        copies = []
        for j in range(NSTREAM):
            copies.append(
                pltpu.async_copy(
                    data_hbm.at[idxb.at[pl.ds(j * 128, 128)]],
                    gbuf.at[pl.ds(j * 128, 128)],
                    sem,
                )
            )
        for c in copies:
            c.wait()

        @pl.loop(0, NSG)
        def _(sg):
            base = sg * 128
            acc = [None] * CH
            for r in range(8):
                wa = wbuf[pl.ds(2 * (base + r * 16), 16)]
                wb = wbuf[pl.ds(2 * (base + r * 16) + 16, 16)]
                kvec = iota16 + (base + r * 16)
                for c in range(CH):
                    cola = plsc.load_gather(gbuf, [kvec, cconst[c]])
                    colb = plsc.load_gather(gbuf, [kvec, cconst[CH + c]])
                    term = wa * cola + wb * colb
                    acc[c] = term if acc[c] is None else acc[c] + term
            for c in range(CH):
                outc[c, pl.ds(sg * 16, 16)] = acc[c]

        for c in range(CH):
            pltpu.sync_copy(outc.at[c], out_hbm.at[c, pl.ds(start, B)])


def _repack_body(in_ref, out_ref):
    inb = in_ref[0, 0]
    for zh in range(8):
        sub = inb[:, :, zh * 8:(zh + 1) * 8]
        v = jnp.transpose(sub, (2, 0, 1))
        vshift = jnp.concatenate([v[:, 1:, :], v[:, 31:32, :]], axis=1)
        vs = jnp.concatenate([v, vshift], axis=2)
        row = jnp.concatenate([vs[zl] for zl in range(8)], axis=1)
        out_ref[pl.ds(zh * 32, 32), :] = row


def _repack(dp):
    out2 = pl.pallas_call(
        _repack_body,
        out_shape=jax.ShapeDtypeStruct((1048576, 128), jnp.float32),
        grid=(64, 64),
        in_specs=[
            pl.BlockSpec((1, 1, 32, 8, 64), lambda x, y: (x, y, 0, 0, 0))
        ],
        out_specs=pl.BlockSpec((256, 128), lambda x, y: (x * 64 + y, 0)),
    )(dp)
    return out2.reshape(-1).reshape(-1, 2 * CH)


def kernel(xs, data):
    xs_t = xs.T
    dp = jnp.transpose(data, (0, 1, 3, 4, 2))
    data2 = _repack(dp)
    mesh = plsc.VectorSubcoreMesh(core_axis_name="c", subcore_axis_name="s")
    cp = pltpu.CompilerParams()
    if "needs_layout_passes" in pltpu.CompilerParams.__dataclass_fields__:
        cp = dataclasses.replace(cp, needs_layout_passes=False)
    if "use_tc_tiling_on_sc" in pltpu.CompilerParams.__dataclass_fields__:
        cp = dataclasses.replace(cp, use_tc_tiling_on_sc=False)
    k = pl.kernel(
        _body,
        out_type=jax.ShapeDtypeStruct((CH, N), jnp.float32),
        mesh=mesh,
        scratch_types=[
            pltpu.VMEM((4, B), jnp.float32),
            pltpu.VMEM((NROW,), jnp.int32),
            pltpu.VMEM((2 * NROW,), jnp.float32),
            pltpu.VMEM((NROW, 2 * CH), jnp.float32),
            pltpu.VMEM((CH, B), jnp.float32),
            pltpu.SemaphoreType.DMA,
        ],
        compiler_params=cp,
    )
    out = k(xs_t, data2)
    return out.T


# speedup vs baseline: 2.7958x; 2.7958x over previous
import dataclasses

import jax
import jax.numpy as jnp
from jax import lax
from jax.experimental import pallas as pl
from jax.experimental.pallas import tpu as pltpu
from jax.experimental.pallas import tpu_sc as plsc

N = 1048576
CH = 8
DIMS = (64, 64, 64, 32)
STRIDES = (DIMS[1] * DIMS[2] * DIMS[3], DIMS[2] * DIMS[3], DIMS[3], 1)
NW = 32
PPW = N // NW
B = 128
NBLK = PPW // B
NSG = B // 16
NROW = 8 * B
NSTREAM = NROW // 128


def _body(xs_hbm, data_hbm, out_hbm, xsb, idxb, wbuf, gbuf, outc, sem):
    wid = lax.axis_index("s") * 2 + lax.axis_index("c")
    base_pt = wid * PPW

    iota16 = lax.iota(jnp.int32, 16)
    cconst = [jnp.full((16,), c, jnp.int32) for c in range(2 * CH)]

    @pl.loop(0, NBLK)
    def _(blk):
        start = base_pt + blk * B
        for d in range(4):
            pltpu.sync_copy(xs_hbm.at[d, pl.ds(start, B)], xsb.at[d])

        @pl.loop(0, NSG)
        def _(sg):
            lo = []
            hi = []
            f = []
            for d in range(4):
                cd = xsb[d, pl.ds(sg * 16, 16)]
                half = 0.5 * DIMS[d]
                sd = cd * half + half
                idx_i = sd.astype(jnp.int32)
                f.append(sd - idx_i.astype(jnp.float32))
                lod = jnp.minimum(jnp.maximum(idx_i, 0), DIMS[d] - 1)
                lo.append(lod)
                hi.append(jnp.minimum(lod + 1, DIMS[d] - 1))
            px = [lo[0] * STRIDES[0], hi[0] * STRIDES[0]]
            py = [lo[1] * STRIDES[1], hi[1] * STRIDES[1]]
            pz = [lo[2] * STRIDES[2], hi[2] * STRIDES[2]]
            t0 = lo[3]
            wx = [1.0 - f[0], f[0]]
            wy = [1.0 - f[1], f[1]]
            wz = [1.0 - f[2], f[2]]
            wt = [1.0 - f[3], f[3]]
            pxy = {}
            wxy = {}
            for bx in range(2):
                for by in range(2):
                    pxy[(bx, by)] = px[bx] + py[by]
                    wxy[(bx, by)] = wx[bx] * wy[by]
            for b in range(8):
                bx, by, bz = b & 1, (b >> 1) & 1, (b >> 2) & 1
                wxyz = wxy[(bx, by)] * wz[bz]
                ofs = sg * 128 + b * 16
                idxb[pl.ds(ofs, 16)] = pxy[(bx, by)] + pz[bz] + t0
                wbuf[pl.ds(2 * ofs, 16)] = wxyz * wt[0]
                wbuf[pl.ds(2 * ofs + 16, 16)] = wxyz * wt[1]

        copies = []
        for j in range(NSTREAM):
            copies.append(
                pltpu.async_copy(
                    data_hbm.at[idxb.at[pl.ds(j * 128, 128)]],
                    gbuf.at[pl.ds(j * 128, 128)],
                    sem,
                )
            )
        for c in copies:
            c.wait()

        @pl.loop(0, NSG)
        def _(sg):
            base = sg * 128
            acc = [None] * CH
            for r in range(8):
                wa = wbuf[pl.ds(2 * (base + r * 16), 16)]
                wb = wbuf[pl.ds(2 * (base + r * 16) + 16, 16)]
                kvec = iota16 + (base + r * 16)
                for c in range(CH):
                    cola = plsc.load_gather(gbuf, [kvec, cconst[c]])
                    colb = plsc.load_gather(gbuf, [kvec, cconst[CH + c]])
                    term = wa * cola + wb * colb
                    acc[c] = term if acc[c] is None else acc[c] + term
            for c in range(CH):
                outc[c, pl.ds(sg * 16, 16)] = acc[c]

        for c in range(CH):
            pltpu.sync_copy(outc.at[c], out_hbm.at[c, pl.ds(start, B)])


def _repack_body(in_ref, out_ref):
    inb = in_ref[0, 0]
    ishift = jnp.concatenate([inb[1:], inb[31:32]], axis=0)
    ip = jnp.concatenate([inb, ishift], axis=1)
    tr = jnp.transpose(ip.reshape(512, 64))
    parts = [tr[:, j * 128:(j + 1) * 128][:, None, :] for j in range(4)]
    out_ref[...] = jnp.concatenate(parts, axis=1).reshape(256, 128)


def _repack(dp):
    out2 = pl.pallas_call(
        _repack_body,
        out_shape=jax.ShapeDtypeStruct((1048576, 128), jnp.float32),
        grid=(64, 64),
        in_specs=[
            pl.BlockSpec((1, 1, 32, 8, 64), lambda x, y: (x, y, 0, 0, 0))
        ],
        out_specs=pl.BlockSpec((256, 128), lambda x, y: (x * 64 + y, 0)),
    )(dp)
    return out2.reshape(-1).reshape(-1, 2 * CH)


def kernel(xs, data):
    xs_t = xs.T
    dp = jnp.transpose(data, (0, 1, 3, 4, 2))
    data2 = _repack(dp)
    mesh = plsc.VectorSubcoreMesh(core_axis_name="c", subcore_axis_name="s")
    cp = pltpu.CompilerParams()
    if "needs_layout_passes" in pltpu.CompilerParams.__dataclass_fields__:
        cp = dataclasses.replace(cp, needs_layout_passes=False)
    if "use_tc_tiling_on_sc" in pltpu.CompilerParams.__dataclass_fields__:
        cp = dataclasses.replace(cp, use_tc_tiling_on_sc=False)
    k = pl.kernel(
        _body,
        out_type=jax.ShapeDtypeStruct((CH, N), jnp.float32),
        mesh=mesh,
        scratch_types=[
            pltpu.VMEM((4, B), jnp.float32),
            pltpu.VMEM((NROW,), jnp.int32),
            pltpu.VMEM((2 * NROW,), jnp.float32),
            pltpu.VMEM((NROW, 2 * CH), jnp.float32),
            pltpu.VMEM((CH, B), jnp.float32),
            pltpu.SemaphoreType.DMA,
        ],
        compiler_params=cp,
    )
    out = k(xs_t, data2)
    return out.T


# speedup vs baseline: 4.3311x; 1.5491x over previous
import dataclasses

import jax
import jax.numpy as jnp
from jax import lax
from jax.experimental import pallas as pl
from jax.experimental.pallas import tpu as pltpu
from jax.experimental.pallas import tpu_sc as plsc

N = 1048576
CH = 8
DIMS = (64, 64, 64, 32)
STRIDES = (DIMS[1] * DIMS[2] * DIMS[3], DIMS[2] * DIMS[3], DIMS[3], 1)
NW = 32
PPW = N // NW
B = 128
NBLK = PPW // B
NSG = B // 16
NROW = 8 * B
NSTREAM = NROW // 128


def _body(xs_hbm, data_hbm, out_hbm, xsb, idxb, wbuf, gbuf, outc, sem):
    wid = lax.axis_index("s") * 2 + lax.axis_index("c")
    base_pt = wid * PPW

    iota16 = lax.iota(jnp.int32, 16)
    cconst = [jnp.full((16,), c, jnp.int32) for c in range(2 * CH)]

    @pl.loop(0, NBLK)
    def _(blk):
        start = base_pt + blk * B
        for d in range(4):
            pltpu.sync_copy(xs_hbm.at[d, pl.ds(start, B)], xsb.at[d])

        @pl.loop(0, NSG)
        def _(sg):
            lo = []
            hi = []
            f = []
            for d in range(4):
                cd = xsb[d, pl.ds(sg * 16, 16)]
                half = 0.5 * DIMS[d]
                sd = cd * half + half
                idx_i = sd.astype(jnp.int32)
                f.append(sd - idx_i.astype(jnp.float32))
                lod = jnp.minimum(jnp.maximum(idx_i, 0), DIMS[d] - 1)
                lo.append(lod)
                hi.append(jnp.minimum(lod + 1, DIMS[d] - 1))
            px = [lo[0] * STRIDES[0], hi[0] * STRIDES[0]]
            py = [lo[1] * STRIDES[1], hi[1] * STRIDES[1]]
            pz = [lo[2] * STRIDES[2], hi[2] * STRIDES[2]]
            t0 = lo[3]
            wx = [1.0 - f[0], f[0]]
            wy = [1.0 - f[1], f[1]]
            wz = [1.0 - f[2], f[2]]
            wt = [1.0 - f[3], f[3]]
            pxy = {}
            wxy = {}
            for bx in range(2):
                for by in range(2):
                    pxy[(bx, by)] = px[bx] + py[by]
                    wxy[(bx, by)] = wx[bx] * wy[by]
            for b in range(8):
                bx, by, bz = b & 1, (b >> 1) & 1, (b >> 2) & 1
                wxyz = wxy[(bx, by)] * wz[bz]
                ofs = sg * 128 + b * 16
                idxb[pl.ds(ofs, 16)] = pxy[(bx, by)] + pz[bz] + t0
                wbuf[pl.ds(2 * ofs, 16)] = wxyz * wt[0]
                wbuf[pl.ds(2 * ofs + 16, 16)] = wxyz * wt[1]

        copies = []
        for j in range(NSTREAM):
            copies.append(
                pltpu.async_copy(
                    data_hbm.at[idxb.at[pl.ds(j * 128, 128)]],
                    gbuf.at[pl.ds(j * 128, 128)],
                    sem,
                )
            )
        for c in copies:
            c.wait()

        @pl.loop(0, NSG)
        def _(sg):
            base = sg * 128
            acc = [None] * CH
            for r in range(8):
                wa = wbuf[pl.ds(2 * (base + r * 16), 16)]
                wb = wbuf[pl.ds(2 * (base + r * 16) + 16, 16)]
                kvec = iota16 + (base + r * 16)
                for c in range(CH):
                    cola = plsc.load_gather(gbuf, [kvec, cconst[c]])
                    colb = plsc.load_gather(gbuf, [kvec, cconst[CH + c]])
                    term = wa * cola + wb * colb
                    acc[c] = term if acc[c] is None else acc[c] + term
            for c in range(CH):
                outc[c, pl.ds(sg * 16, 16)] = acc[c]

        for c in range(CH):
            pltpu.sync_copy(outc.at[c], out_hbm.at[c, pl.ds(start, B)])


def _repack_body(in_ref, out_ref):
    def one_y(y, _):
        inb = in_ref[0, y]
        ishift = jnp.concatenate([inb[1:], inb[31:32]], axis=0)
        ip = jnp.concatenate([inb, ishift], axis=1)
        tr = jnp.transpose(ip.reshape(512, 64))
        parts = [tr[:, j * 128:(j + 1) * 128][:, None, :] for j in range(4)]
        out_ref[pl.ds(y * 256, 256), :] = (
            jnp.concatenate(parts, axis=1).reshape(256, 128)
        )
        return _

    lax.fori_loop(0, 16, one_y, None)


def _repack(dp):
    out2 = pl.pallas_call(
        _repack_body,
        out_shape=jax.ShapeDtypeStruct((1048576, 128), jnp.float32),
        grid=(256,),
        in_specs=[
            pl.BlockSpec((1, 16, 32, 8, 64), lambda xq: (xq // 4, xq % 4, 0, 0, 0))
        ],
        out_specs=pl.BlockSpec((4096, 128), lambda xq: (xq, 0)),
    )(dp)
    return out2.reshape(-1).reshape(-1, 2 * CH)


def kernel(xs, data):
    xs_t = xs.T
    dp = jnp.transpose(data, (0, 1, 3, 4, 2))
    data2 = _repack(dp)
    mesh = plsc.VectorSubcoreMesh(core_axis_name="c", subcore_axis_name="s")
    cp = pltpu.CompilerParams()
    if "needs_layout_passes" in pltpu.CompilerParams.__dataclass_fields__:
        cp = dataclasses.replace(cp, needs_layout_passes=False)
    if "use_tc_tiling_on_sc" in pltpu.CompilerParams.__dataclass_fields__:
        cp = dataclasses.replace(cp, use_tc_tiling_on_sc=False)
    k = pl.kernel(
        _body,
        out_type=jax.ShapeDtypeStruct((CH, N), jnp.float32),
        mesh=mesh,
        scratch_types=[
            pltpu.VMEM((4, B), jnp.float32),
            pltpu.VMEM((NROW,), jnp.int32),
            pltpu.VMEM((2 * NROW,), jnp.float32),
            pltpu.VMEM((NROW, 2 * CH), jnp.float32),
            pltpu.VMEM((CH, B), jnp.float32),
            pltpu.SemaphoreType.DMA,
        ],
        compiler_params=cp,
    )
    out = k(xs_t, data2)
    return out.T


# speedup vs baseline: 6.0359x; 1.3936x over previous
import dataclasses

import jax
import jax.numpy as jnp
from jax import lax
from jax.experimental import pallas as pl
from jax.experimental.pallas import tpu as pltpu
from jax.experimental.pallas import tpu_sc as plsc

N = 1048576
CH = 8
DIMS = (64, 64, 64, 32)
STRIDES = (DIMS[1] * DIMS[2] * DIMS[3], DIMS[2] * DIMS[3], DIMS[3], 1)
NW = 32
PPW = N // NW
B = 128
NBLK = PPW // B
NSG = B // 16
NROW = 8 * B
NSTREAM = NROW // 128


def _body(
    xs_hbm, data_hbm, out_hbm,
    xsbA, xsbB, idxA, idxB, wbA, wbB, gA, gB, outA, outB, semA, semB,
):
    wid = lax.axis_index("s") * 2 + lax.axis_index("c")
    base_pt = wid * PPW

    iota16 = lax.iota(jnp.int32, 16)
    cconst = [jnp.full((16,), c, jnp.int32) for c in range(2 * CH)]

    def phase1(start, xsb, idxb, wbuf):
        pltpu.sync_copy(xs_hbm.at[:, pl.ds(start, B)], xsb)

        @pl.loop(0, NSG)
        def _(sg):
            lo = []
            hi = []
            f = []
            for d in range(4):
                cd = xsb[d, pl.ds(sg * 16, 16)]
                half = 0.5 * DIMS[d]
                sd = cd * half + half
                idx_i = sd.astype(jnp.int32)
                f.append(sd - idx_i.astype(jnp.float32))
                lod = jnp.minimum(jnp.maximum(idx_i, 0), DIMS[d] - 1)
                lo.append(lod)
                hi.append(jnp.minimum(lod + 1, DIMS[d] - 1))
            px = [lo[0] * STRIDES[0], hi[0] * STRIDES[0]]
            py = [lo[1] * STRIDES[1], hi[1] * STRIDES[1]]
            pz = [lo[2] * STRIDES[2], hi[2] * STRIDES[2]]
            t0 = lo[3]
            wx = [1.0 - f[0], f[0]]
            wy = [1.0 - f[1], f[1]]
            wz = [1.0 - f[2], f[2]]
            wt = [1.0 - f[3], f[3]]
            for b in range(8):
                bx, by, bz = b & 1, (b >> 1) & 1, (b >> 2) & 1
                wxyz = (wx[bx] * wy[by]) * wz[bz]
                ofs = sg * 128 + b * 16
                idxb[pl.ds(ofs, 16)] = (px[bx] + py[by]) + pz[bz] + t0
                wbuf[pl.ds(2 * ofs, 16)] = wxyz * wt[0]
                wbuf[pl.ds(2 * ofs + 16, 16)] = wxyz * wt[1]

    def fire(idxb, gbuf, sem):
        for j in range(NSTREAM):
            pltpu.async_copy(
                data_hbm.at[idxb.at[pl.ds(j * 128, 128)]],
                gbuf.at[pl.ds(j * 128, 128)],
                sem,
            )

    def drain(idxb, gbuf, sem):
        for j in range(NSTREAM):
            pltpu.make_async_copy(
                data_hbm.at[idxb.at[pl.ds(j * 128, 128)]],
                gbuf.at[pl.ds(j * 128, 128)],
                sem,
            ).wait()

    def phase3(start, wbuf, gbuf, outc):
        @pl.loop(0, NSG)
        def _(sg):
            base = sg * 128
            acc = [None] * CH
            for r in range(8):
                wa = wbuf[pl.ds(2 * (base + r * 16), 16)]
                wb = wbuf[pl.ds(2 * (base + r * 16) + 16, 16)]
                kvec = iota16 + (base + r * 16)
                for c in range(CH):
                    cola = plsc.load_gather(gbuf, [kvec, cconst[c]])
                    colb = plsc.load_gather(gbuf, [kvec, cconst[CH + c]])
                    term = wa * cola + wb * colb
                    acc[c] = term if acc[c] is None else acc[c] + term
            for c in range(CH):
                outc[c, pl.ds(sg * 16, 16)] = acc[c]

        pltpu.sync_copy(outc, out_hbm.at[:, pl.ds(start, B)])

    phase1(base_pt, xsbA, idxA, wbA)
    fire(idxA, gA, semA)

    @pl.loop(0, NBLK, step=2)
    def _(blk):
        s0 = base_pt + blk * B
        s1 = s0 + B
        phase1(s1, xsbB, idxB, wbB)
        fire(idxB, gB, semB)
        drain(idxA, gA, semA)
        phase3(s0, wbA, gA, outA)

        @pl.when(blk + 2 < NBLK)
        def _():
            phase1(s1 + B, xsbA, idxA, wbA)
            fire(idxA, gA, semA)

        drain(idxB, gB, semB)
        phase3(s1, wbB, gB, outB)


def _repack_body(in_ref, out_ref):
    def one_y(y, _):
        inb = in_ref[0, y]
        ishift = jnp.concatenate([inb[1:], inb[31:32]], axis=0)
        ip = jnp.concatenate([inb, ishift], axis=1)
        tr = jnp.transpose(ip.reshape(512, 64))
        parts = [tr[:, j * 128:(j + 1) * 128][:, None, :] for j in range(4)]
        out_ref[pl.ds(y * 256, 256), :] = (
            jnp.concatenate(parts, axis=1).reshape(256, 128)
        )
        return _

    lax.fori_loop(0, 16, one_y, None)


def _repack(dp):
    out2 = pl.pallas_call(
        _repack_body,
        out_shape=jax.ShapeDtypeStruct((1048576, 128), jnp.float32),
        grid=(256,),
        in_specs=[
            pl.BlockSpec((1, 16, 32, 8, 64), lambda xq: (xq // 4, xq % 4, 0, 0, 0))
        ],
        out_specs=pl.BlockSpec((4096, 128), lambda xq: (xq, 0)),
    )(dp)
    return out2.reshape(-1).reshape(-1, 2 * CH)


def kernel(xs, data):
    xs_t = xs.T
    dp = jnp.transpose(data, (0, 1, 3, 4, 2))
    data2 = _repack(dp)
    mesh = plsc.VectorSubcoreMesh(core_axis_name="c", subcore_axis_name="s")
    cp = pltpu.CompilerParams()
    if "needs_layout_passes" in pltpu.CompilerParams.__dataclass_fields__:
        cp = dataclasses.replace(cp, needs_layout_passes=False)
    if "use_tc_tiling_on_sc" in pltpu.CompilerParams.__dataclass_fields__:
        cp = dataclasses.replace(cp, use_tc_tiling_on_sc=False)
    k = pl.kernel(
        _body,
        out_type=jax.ShapeDtypeStruct((CH, N), jnp.float32),
        mesh=mesh,
        scratch_types=[
            pltpu.VMEM((4, B), jnp.float32),
            pltpu.VMEM((4, B), jnp.float32),
            pltpu.VMEM((NROW,), jnp.int32),
            pltpu.VMEM((NROW,), jnp.int32),
            pltpu.VMEM((2 * NROW,), jnp.float32),
            pltpu.VMEM((2 * NROW,), jnp.float32),
            pltpu.VMEM((NROW, 2 * CH), jnp.float32),
            pltpu.VMEM((NROW, 2 * CH), jnp.float32),
            pltpu.VMEM((CH, B), jnp.float32),
            pltpu.VMEM((CH, B), jnp.float32),
            pltpu.SemaphoreType.DMA,
            pltpu.SemaphoreType.DMA,
        ],
        compiler_params=cp,
    )
    out = k(xs_t, data2)
    return out.T


# speedup vs baseline: 8.9558x; 1.4837x over previous
import dataclasses

import jax
import jax.numpy as jnp
from jax import lax
from jax.experimental import pallas as pl
from jax.experimental.pallas import tpu as pltpu
from jax.experimental.pallas import tpu_sc as plsc

N = 1048576
CH = 8
DIMS = (64, 64, 64, 32)
STRIDES = (DIMS[1] * DIMS[2] * DIMS[3], DIMS[2] * DIMS[3], DIMS[3], 1)
NW = 32
PPW = N // NW
B = 128
NBLK = PPW // B
NSG = B // 16
NROW = 8 * B
NSTREAM = NROW // 128


def _body(
    xs_hbm, data_hbm, out_hbm,
    xsbA, xsbB, idxA, idxB, wbA, wbB, gA, gB, outA, outB, semA, semB,
):
    wid = lax.axis_index("s") * 2 + lax.axis_index("c")
    base_pt = wid * PPW

    iota16 = lax.iota(jnp.int32, 16)
    cconst = [jnp.full((16,), c, jnp.int32) for c in range(2 * CH)]

    def phase1(start, xsb, idxb, wbuf):
        pltpu.sync_copy(xs_hbm.at[start // B], xsb)

        @pl.loop(0, NSG)
        def _(sg):
            lo = []
            hi = []
            f = []
            for d in range(4):
                cd = xsb[d, pl.ds(sg * 16, 16)]
                half = 0.5 * DIMS[d]
                sd = cd * half + half
                idx_i = sd.astype(jnp.int32)
                f.append(sd - idx_i.astype(jnp.float32))
                lod = jnp.minimum(jnp.maximum(idx_i, 0), DIMS[d] - 1)
                lo.append(lod)
                hi.append(jnp.minimum(lod + 1, DIMS[d] - 1))
            px = [lo[0] * STRIDES[0], hi[0] * STRIDES[0]]
            py = [lo[1] * STRIDES[1], hi[1] * STRIDES[1]]
            pz = [lo[2] * STRIDES[2], hi[2] * STRIDES[2]]
            t0 = lo[3]
            wx = [1.0 - f[0], f[0]]
            wy = [1.0 - f[1], f[1]]
            wz = [1.0 - f[2], f[2]]
            wt = [1.0 - f[3], f[3]]
            for b in range(8):
                bx, by, bz = b & 1, (b >> 1) & 1, (b >> 2) & 1
                wxyz = (wx[bx] * wy[by]) * wz[bz]
                ofs = sg * 128 + b * 16
                idxb[pl.ds(ofs, 16)] = (px[bx] + py[by]) + pz[bz] + t0
                wbuf[pl.ds(2 * ofs, 16)] = wxyz * wt[0]
                wbuf[pl.ds(2 * ofs + 16, 16)] = wxyz * wt[1]

    def fire(idxb, gbuf, sem):
        for j in range(NSTREAM):
            pltpu.async_copy(
                data_hbm.at[idxb.at[pl.ds(j * 128, 128)]],
                gbuf.at[pl.ds(j * 128, 128)],
                sem,
            )

    def drain(idxb, gbuf, sem):
        for j in range(NSTREAM):
            pltpu.make_async_copy(
                data_hbm.at[idxb.at[pl.ds(j * 128, 128)]],
                gbuf.at[pl.ds(j * 128, 128)],
                sem,
            ).wait()

    def phase3(start, wbuf, gbuf, outc):
        @pl.loop(0, NSG)
        def _(sg):
            base = sg * 128
            acc = [None] * CH
            for r in range(8):
                wa = wbuf[pl.ds(2 * (base + r * 16), 16)]
                wb = wbuf[pl.ds(2 * (base + r * 16) + 16, 16)]
                kvec = iota16 + (base + r * 16)
                for c in range(CH):
                    cola = plsc.load_gather(gbuf, [kvec, cconst[c]])
                    colb = plsc.load_gather(gbuf, [kvec, cconst[CH + c]])
                    term = wa * cola + wb * colb
                    acc[c] = term if acc[c] is None else acc[c] + term
            for c in range(CH):
                outc[c, pl.ds(sg * 16, 16)] = acc[c]

        pltpu.sync_copy(outc, out_hbm.at[start // B])

    phase1(base_pt, xsbA, idxA, wbA)
    fire(idxA, gA, semA)

    @pl.loop(0, NBLK, step=2)
    def _(blk):
        s0 = base_pt + blk * B
        s1 = s0 + B
        phase1(s1, xsbB, idxB, wbB)
        fire(idxB, gB, semB)
        drain(idxA, gA, semA)
        phase3(s0, wbA, gA, outA)

        @pl.when(blk + 2 < NBLK)
        def _():
            phase1(s1 + B, xsbA, idxA, wbA)
            fire(idxA, gA, semA)

        drain(idxB, gB, semB)
        phase3(s1, wbB, gB, outB)


def _repack_body(in_ref, out_ref):
    def one_y(y, _):
        inb = in_ref[0, y]
        ishift = jnp.concatenate([inb[1:], inb[31:32]], axis=0)
        ip = jnp.concatenate([inb, ishift], axis=1)
        tr = jnp.transpose(ip.reshape(512, 64))
        parts = [tr[:, j * 128:(j + 1) * 128][:, None, :] for j in range(4)]
        out_ref[pl.ds(y * 256, 256), :] = (
            jnp.concatenate(parts, axis=1).reshape(256, 128)
        )
        return _

    lax.fori_loop(0, 16, one_y, None)


def _repack(dp):
    out2 = pl.pallas_call(
        _repack_body,
        out_shape=jax.ShapeDtypeStruct((1048576, 128), jnp.float32),
        grid=(256,),
        in_specs=[
            pl.BlockSpec((1, 16, 32, 8, 64), lambda xq: (xq // 4, xq % 4, 0, 0, 0))
        ],
        out_specs=pl.BlockSpec((4096, 128), lambda xq: (xq, 0)),
    )(dp)
    return out2.reshape(-1).reshape(-1, 2 * CH)


def kernel(xs, data):
    xs_t = jnp.transpose(xs.reshape(N // B, B, 4), (0, 2, 1))
    dp = jnp.transpose(data, (0, 1, 3, 4, 2))
    data2 = _repack(dp)
    mesh = plsc.VectorSubcoreMesh(core_axis_name="c", subcore_axis_name="s")
    cp = pltpu.CompilerParams()
    if "needs_layout_passes" in pltpu.CompilerParams.__dataclass_fields__:
        cp = dataclasses.replace(cp, needs_layout_passes=False)
    if "use_tc_tiling_on_sc" in pltpu.CompilerParams.__dataclass_fields__:
        cp = dataclasses.replace(cp, use_tc_tiling_on_sc=False)
    k = pl.kernel(
        _body,
        out_type=jax.ShapeDtypeStruct((N // B, CH, B), jnp.float32),
        mesh=mesh,
        scratch_types=[
            pltpu.VMEM((4, B), jnp.float32),
            pltpu.VMEM((4, B), jnp.float32),
            pltpu.VMEM((NROW,), jnp.int32),
            pltpu.VMEM((NROW,), jnp.int32),
            pltpu.VMEM((2 * NROW,), jnp.float32),
            pltpu.VMEM((2 * NROW,), jnp.float32),
            pltpu.VMEM((NROW, 2 * CH), jnp.float32),
            pltpu.VMEM((NROW, 2 * CH), jnp.float32),
            pltpu.VMEM((CH, B), jnp.float32),
            pltpu.VMEM((CH, B), jnp.float32),
            pltpu.SemaphoreType.DMA,
            pltpu.SemaphoreType.DMA,
        ],
        compiler_params=cp,
    )
    out = k(xs_t, data2)
    return jnp.transpose(out, (0, 2, 1)).reshape(N, CH)
